# baseline (speedup 1.0000x reference)
import functools

import jax
import jax.numpy as jnp
from jax import lax
from jax.experimental import pallas as pl
from jax.experimental.pallas import tpu as pltpu, tpu_sc as plsc

_info = plsc.get_sparse_core_info()
_NC = _info.num_cores
_NS = _info.num_subcores
_NW = _NC * _NS

_CHUNK = 256


@functools.lru_cache(maxsize=None)
def _make_gather(ns: int, nb: int, D: int):
    assert D == 64
    b_per_w = nb // _NW
    hpw = b_per_w // _CHUNK
    n_chunks = ns * hpw

    mesh = plsc.VectorSubcoreMesh(core_axis_name="c", subcore_axis_name="s")

    @functools.partial(
        pl.kernel,
        out_type=jax.ShapeDtypeStruct((ns, D, nb), jnp.float32),
        mesh=mesh,
        scratch_types=(
            [pltpu.VMEM((_CHUNK,), jnp.int32) for _ in range(2)]
            + [pltpu.VMEM((_CHUNK,), jnp.int32) for _ in range(2)]
            + [pltpu.VMEM((_CHUNK, 2 * D), jnp.float32) for _ in range(2)]
            + [pltpu.VMEM((D, _CHUNK), jnp.float32) for _ in range(2)]
            + [pltpu.SemaphoreType.DMA for _ in range(4)]
        ),
        compiler_params=pltpu.CompilerParams(
            use_tc_tiling_on_sc=True, needs_layout_passes=False
        ),
    )
    def gather_kernel(table2_hbm, ih_hbm, io_hbm, out_hbm, *refs):
        ih = refs[0:2]
        io = refs[2:4]
        rows = refs[4:6]
        tb = refs[6:8]
        gsem = refs[8:10]
        osem = refs[10:12]

        wid = lax.axis_index("s") * _NC + lax.axis_index("c")
        bbase = wid * b_per_w

        iota = lax.iota(jnp.int32, 16)

        def split(c):
            s = c // hpw
            b0 = bbase + (c % hpw) * _CHUNK
            return s, b0

        def stage_idx(c, b):
            s, b0 = split(c)
            p0 = pl.multiple_of(s * nb + b0, _CHUNK)
            pltpu.sync_copy(ih_hbm.at[pl.ds(p0, _CHUNK)], ih[b])
            pltpu.sync_copy(io_hbm.at[pl.ds(p0, _CHUNK)], io[b])

        def gather_copy(b):
            return pltpu.make_async_copy(table2_hbm.at[ih[b]], rows[b], gsem[b])

        def out_copy(c, b):
            s, b0 = split(c)
            return pltpu.make_async_copy(
                tb[b],
                out_hbm.at[s, :, pl.ds(pl.multiple_of(b0, _CHUNK), _CHUNK)],
                osem[b],
            )

        stage_idx(0, 0)
        gather_copy(0).start()

        def step(c, b):
            @pl.when(c + 1 < n_chunks)
            def _():
                stage_idx(c + 1, 1 - b)
                gather_copy(1 - b).start()

            gather_copy(b).wait()

            @pl.when(c >= 2)
            def _():
                out_copy(c - 2, b).wait()

            def blk(R, carry):
                rr = R * 16 + iota
                iov = io[b][pl.ds(pl.multiple_of(R * 16, 16), 16)]
                for j in range(0, 16, 2):
                    t0 = (iota + j) & 15
                    t1 = (iota + (j + 1)) & 15
                    sc0 = iov + t0
                    sc1 = iov + t1
                    vals = [
                        plsc.load_gather(rows[b], [rr, sc + (16 * C)])
                        for sc in (sc0, sc1)
                        for C in range(4)
                    ]
                    for k, (t, C) in enumerate(
                        [(t, C) for t in (t0, t1) for C in range(4)]
                    ):
                        plsc.store_scatter(tb[b], [t + (16 * C), rr], vals[k])
                return carry

            lax.fori_loop(0, _CHUNK // 16, blk, 0)
            out_copy(c, b).start()

        def pair(g, carry):
            step(2 * g, 0)
            step(2 * g + 1, 1)
            return carry

        lax.fori_loop(0, n_chunks // 2, pair, 0)

        out_copy(n_chunks - 2, 0).wait()
        out_copy(n_chunks - 1, 1).wait()

    return gather_kernel


def kernel(indices, table):
    nb, ns = indices.shape
    V, D = table.shape
    half = V // 2
    flat = indices.T.reshape(nb * ns).astype(jnp.int32)
    table2 = table.reshape(half, 2 * D)
    ih = flat >> 1
    io = (flat & 1) << 6
    out = _make_gather(ns, nb, D)(table2, ih, io)
    return out.transpose(2, 0, 1)


# speedup vs baseline: 1.1157x; 1.1157x over previous
import functools

import jax
import jax.numpy as jnp
from jax import lax
from jax.experimental import pallas as pl
from jax.experimental.pallas import tpu as pltpu, tpu_sc as plsc

_info = plsc.get_sparse_core_info()
_NC = _info.num_cores
_NS = _info.num_subcores
_NW = _NC * _NS

_CHUNK = 256


@functools.lru_cache(maxsize=None)
def _make_gather(ns: int, nb: int, D: int):
    assert D == 64
    b_per_w = nb // _NW
    hpw = b_per_w // _CHUNK
    n_chunks = ns * hpw

    mesh = plsc.VectorSubcoreMesh(core_axis_name="c", subcore_axis_name="s")

    @functools.partial(
        pl.kernel,
        out_type=jax.ShapeDtypeStruct((ns, D, nb), jnp.float32),
        mesh=mesh,
        scratch_types=(
            [pltpu.VMEM((_CHUNK,), jnp.int32) for _ in range(2)]
            + [pltpu.VMEM((_CHUNK,), jnp.int32) for _ in range(2)]
            + [pltpu.VMEM((_CHUNK, 2 * D), jnp.float32) for _ in range(2)]
            + [pltpu.VMEM((D, _CHUNK), jnp.float32) for _ in range(2)]
            + [pltpu.SemaphoreType.DMA for _ in range(4)]
        ),
        compiler_params=pltpu.CompilerParams(
            use_tc_tiling_on_sc=True, needs_layout_passes=False
        ),
    )
    def gather_kernel(table2_hbm, ih_hbm, io_hbm, out_hbm, *refs):
        ih = refs[0:2]
        io = refs[2:4]
        rows = refs[4:6]
        tb = refs[6:8]
        gsem = refs[8:10]
        osem = refs[10:12]

        wid = lax.axis_index("s") * _NC + lax.axis_index("c")
        bbase = wid * b_per_w

        iota = lax.iota(jnp.int32, 16)

        def split(c):
            s = c // hpw
            b0 = bbase + (c % hpw) * _CHUNK
            return s, b0

        def stage_idx(c, b):
            s, b0 = split(c)
            p0 = pl.multiple_of(s * nb + b0, _CHUNK)
            pltpu.sync_copy(ih_hbm.at[pl.ds(p0, _CHUNK)], ih[b])
            pltpu.sync_copy(io_hbm.at[pl.ds(p0, _CHUNK)], io[b])

        def gather_copy(b):
            return pltpu.make_async_copy(table2_hbm.at[ih[b]], rows[b], gsem[b])

        def out_copy(c, b):
            s, b0 = split(c)
            return pltpu.make_async_copy(
                tb[b],
                out_hbm.at[s, :, pl.ds(pl.multiple_of(b0, _CHUNK), _CHUNK)],
                osem[b],
            )

        stage_idx(0, 0)
        gather_copy(0).start()

        def step(c, b):
            @pl.when(c + 1 < n_chunks)
            def _():
                stage_idx(c + 1, 1 - b)
                gather_copy(1 - b).start()

            gather_copy(b).wait()

            @pl.when(c >= 2)
            def _():
                out_copy(c - 2, b).wait()

            @plsc.parallel_loop(0, _CHUNK // 16, unroll=2)
            def blk(R):
                rr = R * 16 + iota
                iov = io[b][pl.ds(pl.multiple_of(R * 16, 16), 16)]
                for j in range(16):
                    t = (iota + j) & 15
                    sc = iov + t
                    vals = [
                        plsc.load_gather(rows[b], [rr, sc + (16 * C)])
                        for C in range(4)
                    ]
                    for C in range(4):
                        plsc.store_scatter(tb[b], [t + (16 * C), rr], vals[C])

            out_copy(c, b).start()

        def pair(g, carry):
            step(2 * g, 0)
            step(2 * g + 1, 1)
            return carry

        lax.fori_loop(0, n_chunks // 2, pair, 0)

        out_copy(n_chunks - 2, 0).wait()
        out_copy(n_chunks - 1, 1).wait()

    return gather_kernel


def kernel(indices, table):
    nb, ns = indices.shape
    V, D = table.shape
    half = V // 2
    flat = indices.T.reshape(nb * ns).astype(jnp.int32)
    table2 = table.reshape(half, 2 * D)
    ih = flat >> 1
    io = (flat & 1) << 6
    out = _make_gather(ns, nb, D)(table2, ih, io)
    return out.transpose(2, 0, 1)
